# Initial kernel scaffold; baseline (speedup 1.0000x reference)
import jax
import jax.numpy as jnp
from jax.experimental import pallas as pl


def kernel(x, edge_index, W1, b1, W2, b2, Wih0, Whh0, bih0, bhh0, Wih1, Whh1, bih1, bhh1, Wout, bout):
    raise NotImplementedError("write your pallas kernel here")


# speedup vs baseline: 59.0986x; 59.0986x over previous
import functools

import jax
import jax.numpy as jnp
from jax import lax
from jax.experimental import pallas as pl
from jax.experimental.pallas import tpu as pltpu
from jax.experimental.pallas import tpu_sc as plsc


def _build_a0(s_all, d_all, np_, chunk, nchunk, rows):
    mesh = plsc.VectorSubcoreMesh(core_axis_name="c", subcore_axis_name="s")
    n_blocks = np_ // rows
    passes = n_blocks // 32

    def body(s_hbm, d_hbm, z_hbm, a0_hbm, block_v, s_v, d_v):
        cid = lax.axis_index("c")
        sid = lax.axis_index("s")
        wid = sid * 2 + cid
        ones = jnp.full((16,), 1.0, jnp.float32)
        for p in range(passes):
            lo = (wid + p * 32) * rows
            pltpu.sync_copy(z_hbm, block_v)
            for ci in range(nchunk):
                pltpu.sync_copy(s_hbm.at[pl.ds(ci * chunk, chunk)], s_v)
                pltpu.sync_copy(d_hbm.at[pl.ds(ci * chunk, chunk)], d_v)

                def step(j, carry):
                    sv = s_v[pl.ds(j * 16, 16)]
                    dv = d_v[pl.ds(j * 16, 16)]
                    rel = dv - lo
                    msk = (rel >= 0) & (rel < rows)
                    idx = jnp.where(msk, rel * np_ + sv, 0)
                    plsc.addupdate_scatter(block_v, [idx], ones, mask=msk)
                    return carry

                lax.fori_loop(0, chunk // 16, step, 0)
            pltpu.sync_copy(block_v, a0_hbm.at[pl.ds(lo * np_, rows * np_)])

    zeros = jnp.zeros((rows * np_,), jnp.float32)
    call = functools.partial(
        pl.kernel,
        out_type=jax.ShapeDtypeStruct((np_ * np_,), jnp.float32),
        mesh=mesh,
        compiler_params=pltpu.CompilerParams(needs_layout_passes=False),
        scratch_types=[
            pltpu.VMEM((rows * np_,), jnp.float32),
            pltpu.VMEM((chunk,), jnp.int32),
            pltpu.VMEM((chunk,), jnp.int32),
        ],
    )(body)
    return call(s_all, d_all, zeros).reshape(np_, np_)


def _dinv_kernel(a0_ref, dinv_ref):
    deg = jnp.sum(a0_ref[...], axis=1, keepdims=True)
    dinv_ref[...] = lax.rsqrt(deg)


def _layer1_kernel(a0_ref, xs_ref, dinv_f_ref, dinv_t_ref, w1_ref, b1_ref,
                   g1_ref):
    xs = xs_ref[...] * dinv_f_ref[...]
    u = jnp.dot(a0_ref[...], xs, preferred_element_type=jnp.float32)
    v = jnp.dot(u * dinv_t_ref[...], w1_ref[...],
                preferred_element_type=jnp.float32) + b1_ref[...]
    g1_ref[...] = jnp.maximum(v, 0.0) * dinv_t_ref[...]


def _layer2_kernel(nz, tm, g_count, dd, a0_ref, g1_ref, dinv_t_ref, w2t_ref,
                   b2_ref, out_ref):
    i = pl.program_id(0)
    y = jnp.dot(a0_ref[...], g1_ref[...], preferred_element_type=jnp.float32)
    z = y * dinv_t_ref[...]
    rows = lax.broadcasted_iota(jnp.int32, (tm, 1), 0) + i * tm
    valid = rows < nz
    parts = []
    for g in range(g_count):
        h2g = jnp.dot(z[:, g * dd:(g + 1) * dd], w2t_ref[...],
                      preferred_element_type=jnp.float32) + b2_ref[...]
        h2g = jnp.maximum(h2g, 0.0)
        parts.append(jnp.sum(jnp.where(valid, h2g, 0.0), axis=0,
                             keepdims=True))
    out_ref[...] = jnp.concatenate(parts, axis=0)[None]


def _lstm_kernel(nz, sz, bz, p_ref, wih0_ref, whh0_ref, bi0_ref, bh0_ref,
                 wih1_ref, whh1_ref, bi1_ref, bh1_ref, wo_ref, bo_ref,
                 out_ref):
    seq = jnp.sum(p_ref[...], axis=0) * (1.0 / nz)
    dd = seq.shape[1]

    def run_layer(get_x, wih, whh, bias):
        h = jnp.zeros((bz, dd), jnp.float32)
        c = jnp.zeros((bz, dd), jnp.float32)
        hs = []
        for t in range(sz):
            xt = get_x(t)
            g = (jnp.dot(xt, wih, preferred_element_type=jnp.float32)
                 + jnp.dot(h, whh, preferred_element_type=jnp.float32)
                 + bias)
            ig = jax.nn.sigmoid(g[:, 0:dd])
            fg = jax.nn.sigmoid(g[:, dd:2 * dd])
            gg = jnp.tanh(g[:, 2 * dd:3 * dd])
            og = jax.nn.sigmoid(g[:, 3 * dd:4 * dd])
            c = fg * c + ig * gg
            h = og * jnp.tanh(c)
            hs.append(h)
        return hs

    def x0(t):
        return jnp.concatenate([seq[b * sz + t:b * sz + t + 1]
                                for b in range(bz)], axis=0)

    hs0 = run_layer(x0, wih0_ref[...], whh0_ref[...],
                    bi0_ref[...] + bh0_ref[...])
    hs1 = run_layer(lambda t: hs0[t], wih1_ref[...], whh1_ref[...],
                    bi1_ref[...] + bh1_ref[...])
    out_ref[...] = jnp.dot(hs1[-1], wo_ref[...],
                           preferred_element_type=jnp.float32) + bo_ref[...]


def kernel(x, edge_index, W1, b1, W2, b2, Wih0, Whh0, bih0, bhh0,
           Wih1, Whh1, bih1, bhh1, Wout, bout):
    bz, sz, nz, fz = x.shape
    g_count = bz * sz
    dd = W1.shape[0]
    np_ = ((nz + 255) // 256) * 256
    tm = 256
    nt = np_ // tm
    e = edge_index.shape[1]

    src = edge_index[0].astype(jnp.int32)
    dst = edge_index[1].astype(jnp.int32)
    loop = jnp.arange(np_, dtype=jnp.int32)
    etot = ((e + np_ + 511) // 512) * 512
    extra = etot - e - np_
    fill = jnp.full((extra,), np_ - 1, jnp.int32)
    s_all = jnp.concatenate([src, loop, fill])
    d_all = jnp.concatenate([dst, loop, fill])

    nchunk = 2
    chunk = etot // nchunk

    a0 = _build_a0(s_all, d_all, np_, chunk, nchunk, rows=32)

    dinv = pl.pallas_call(
        _dinv_kernel,
        grid=(nt,),
        in_specs=[pl.BlockSpec((tm, np_), lambda i: (i, 0))],
        out_specs=pl.BlockSpec((tm, 1), lambda i: (i, 0)),
        out_shape=jax.ShapeDtypeStruct((np_, 1), jnp.float32),
    )(a0)

    xf = x.reshape(g_count, nz, fz)
    xs = jnp.pad(xf.transpose(1, 0, 2).reshape(nz, g_count * fz),
                 ((0, np_ - nz), (0, 0)))
    w1blk = jnp.kron(jnp.eye(g_count, dtype=jnp.float32), W1.T)
    b1blk = jnp.tile(b1, g_count)[None]

    g1 = pl.pallas_call(
        _layer1_kernel,
        grid=(nt,),
        in_specs=[
            pl.BlockSpec((tm, np_), lambda i: (i, 0)),
            pl.BlockSpec((np_, g_count * fz), lambda i: (0, 0)),
            pl.BlockSpec((np_, 1), lambda i: (0, 0)),
            pl.BlockSpec((tm, 1), lambda i: (i, 0)),
            pl.BlockSpec((g_count * fz, np_), lambda i: (0, 0)),
            pl.BlockSpec((1, np_), lambda i: (0, 0)),
        ],
        out_specs=pl.BlockSpec((tm, np_), lambda i: (i, 0)),
        out_shape=jax.ShapeDtypeStruct((np_, np_), jnp.float32),
    )(a0, xs, dinv, dinv, w1blk, b1blk)

    partials = pl.pallas_call(
        functools.partial(_layer2_kernel, nz, tm, g_count, dd),
        grid=(nt,),
        in_specs=[
            pl.BlockSpec((tm, np_), lambda i: (i, 0)),
            pl.BlockSpec((np_, np_), lambda i: (0, 0)),
            pl.BlockSpec((tm, 1), lambda i: (i, 0)),
            pl.BlockSpec((dd, dd), lambda i: (0, 0)),
            pl.BlockSpec((1, dd), lambda i: (0, 0)),
        ],
        out_specs=pl.BlockSpec((1, g_count, dd), lambda i: (i, 0, 0)),
        out_shape=jax.ShapeDtypeStruct((nt, g_count, dd), jnp.float32),
    )(a0, g1, dinv, W2.T, b2[None])

    out = pl.pallas_call(
        functools.partial(_lstm_kernel, nz, sz, bz),
        out_shape=jax.ShapeDtypeStruct((bz, 2), jnp.float32),
    )(partials, Wih0.T, Whh0.T, bih0[None], bhh0[None],
      Wih1.T, Whh1.T, bih1[None], bhh1[None], Wout.T, bout[None])
    return out
